# baseline (speedup 1.0000x reference)
import functools

import jax
import jax.numpy as jnp
from jax import lax
from jax.experimental import pallas as pl
from jax.experimental.pallas import tpu as pltpu
from jax.experimental.pallas import tpu_sc as plsc

V, M, N = 50000, 512, 16384
NC, NS, LANES = 2, 16, 16
NW = NC * NS
B_PER_W = N // NW
CHUNK = 8
NCHUNK = B_PER_W // CHUNK
MV = M // LANES
NBUF = 10


def _sc_body(eig_hbm, table_hbm, idx_hbm, ls_hbm, var_hbm, out_hbm,
             idx_v, s_v, eig_v, ls_s, var_s, bufs, gsem, ssem):
    cid = lax.axis_index("c")
    sid = lax.axis_index("s")
    wid = sid * NC + cid
    base = wid * B_PER_W

    def gather(c, buf_row0):
        pltpu.async_copy(table_hbm.at[idx_v.at[pl.ds(c * CHUNK, CHUNK)]],
                         bufs.at[pl.ds(buf_row0, CHUNK)], gsem)

    def chunk_wait(sem):
        pltpu.make_async_copy(out_hbm.at[pl.ds(0, CHUNK)],
                              bufs.at[pl.ds(0, CHUNK)], sem).wait()

    pltpu.sync_copy(idx_hbm.at[pl.ds(base, CHUNK)], idx_v.at[pl.ds(0, CHUNK)])
    gather(0, 0)
    pltpu.sync_copy(idx_hbm.at[pl.ds(base + CHUNK, B_PER_W - CHUNK)],
                    idx_v.at[pl.ds(CHUNK, B_PER_W - CHUNK)])
    for k in range(1, NBUF - 1):
        gather(k, k * CHUNK)

    pltpu.sync_copy(eig_hbm, eig_v)
    pltpu.sync_copy(ls_hbm, ls_s.at[pl.ds(0, 1)])
    pltpu.sync_copy(var_hbm, var_s.at[pl.ds(0, 1)])

    lsv = jnp.full((LANES,), ls_s[...][0], dtype=jnp.float32)
    var = jnp.full((LANES,), var_s[...][0], dtype=jnp.float32)
    coef = -0.5 / (lsv * lsv)

    def s_body(j, carry):
        sl = pl.ds(j * LANES, LANES)
        s_v[sl] = var * jnp.exp(eig_v[sl] * coef)
        return carry
    lax.fori_loop(0, MV, s_body, 0)

    def chunk_body(c, carry):
        row0 = lax.rem(c, NBUF) * CHUNK
        chunk_wait(gsem)

        s_regs = [s_v[pl.ds(j * LANES, LANES)] for j in range(MV)]

        def row_body(r, rcarry):
            for j in range(MV):
                sl = pl.ds(j * LANES, LANES)
                bufs[row0 + r, sl] = bufs[row0 + r, sl] * s_regs[j]
            return rcarry
        lax.fori_loop(0, CHUNK, row_body, 0)

        pltpu.async_copy(bufs.at[pl.ds(row0, CHUNK)],
                         out_hbm.at[pl.ds(base + c * CHUNK, CHUNK)], ssem)

        @pl.when(jnp.logical_and(c >= 1, c + (NBUF - 1) < NCHUNK))
        def _wait_prev_scatter():
            chunk_wait(ssem)

        @pl.when(c + (NBUF - 1) < NCHUNK)
        def _prefetch():
            gather(c + (NBUF - 1), lax.rem(c + (NBUF - 1), NBUF) * CHUNK)

        return carry

    lax.fori_loop(0, NCHUNK, chunk_body, 0)
    for _ in range(NBUF):
        chunk_wait(ssem)


_mesh = plsc.VectorSubcoreMesh(core_axis_name="c", subcore_axis_name="s")

_sc_kernel = functools.partial(
    pl.kernel,
    mesh=_mesh,
    out_type=jax.ShapeDtypeStruct((N, M), jnp.float32),
    scratch_types=[
        pltpu.VMEM((B_PER_W,), jnp.int32),
        pltpu.VMEM((M,), jnp.float32),
        pltpu.VMEM((M,), jnp.float32),
        pltpu.VMEM((LANES,), jnp.float32),
        pltpu.VMEM((LANES,), jnp.float32),
        pltpu.VMEM((NBUF * CHUNK, M), jnp.float32),
        pltpu.SemaphoreType.DMA,
        pltpu.SemaphoreType.DMA,
    ],
)(_sc_body)


def kernel(eigvals, eigvecs, node_indices, lengthscale, variance):
    idx = node_indices.astype(jnp.int32)
    return _sc_kernel(eigvals, eigvecs, idx, lengthscale, variance)


# speedup vs baseline: 2.4013x; 2.4013x over previous
import functools

import jax
import jax.numpy as jnp
from jax import lax
from jax.experimental import pallas as pl
from jax.experimental.pallas import tpu as pltpu
from jax.experimental.pallas import tpu_sc as plsc

V, M, N = 50000, 512, 16384
NC, NS, LANES = 2, 16, 16
NW = NC * NS
B_PER_W = N // NW
CHUNK = 16
NCHUNK = B_PER_W // CHUNK
MV = M // LANES
NBUF = 6


def _sc_body(eig_hbm, table_hbm, idx_hbm, ls_hbm, var_hbm, out_hbm,
             idx_v, s_v, eig_v, ls_s, var_s, bufs, gsem, ssem):
    cid = lax.axis_index("c")
    sid = lax.axis_index("s")
    wid = sid * NC + cid
    base = wid * B_PER_W

    pltpu.sync_copy(idx_hbm.at[pl.ds(base, B_PER_W)], idx_v)

    def gather(c, buf_row0):
        pltpu.async_copy(table_hbm.at[idx_v.at[pl.ds(c * CHUNK, CHUNK)]],
                         bufs.at[pl.ds(buf_row0, CHUNK)], gsem)

    def chunk_wait(sem):
        pltpu.make_async_copy(out_hbm.at[pl.ds(0, CHUNK)],
                              bufs.at[pl.ds(0, CHUNK)], sem).wait()

    for k in range(NBUF - 1):
        gather(k, k * CHUNK)

    pltpu.sync_copy(eig_hbm, eig_v)
    pltpu.sync_copy(ls_hbm, ls_s.at[pl.ds(0, 1)])
    pltpu.sync_copy(var_hbm, var_s.at[pl.ds(0, 1)])

    lsv = jnp.full((LANES,), ls_s[...][0], dtype=jnp.float32)
    var = jnp.full((LANES,), var_s[...][0], dtype=jnp.float32)
    coef = -0.5 / (lsv * lsv)

    def s_body(j, carry):
        sl = pl.ds(j * LANES, LANES)
        s_v[sl] = var * jnp.exp(eig_v[sl] * coef)
        return carry
    lax.fori_loop(0, MV, s_body, 0)

    def chunk_body(c, carry):
        row0 = lax.rem(c, NBUF) * CHUNK
        chunk_wait(gsem)

        s_regs = [s_v[pl.ds(j * LANES, LANES)] for j in range(MV)]

        def row_body(r, rcarry):
            for j in range(MV):
                sl = pl.ds(j * LANES, LANES)
                bufs[row0 + r, sl] = bufs[row0 + r, sl] * s_regs[j]
            return rcarry
        lax.fori_loop(0, CHUNK, row_body, 0)

        pltpu.async_copy(bufs.at[pl.ds(row0, CHUNK)],
                         out_hbm.at[pl.ds(base + c * CHUNK, CHUNK)], ssem)

        @pl.when(jnp.logical_and(c >= 1, c + (NBUF - 1) < NCHUNK))
        def _wait_prev_scatter():
            chunk_wait(ssem)

        @pl.when(c + (NBUF - 1) < NCHUNK)
        def _prefetch():
            gather(c + (NBUF - 1), lax.rem(c + (NBUF - 1), NBUF) * CHUNK)

        return carry

    lax.fori_loop(0, NCHUNK, chunk_body, 0)
    for _ in range(NBUF):
        chunk_wait(ssem)


_mesh = plsc.VectorSubcoreMesh(core_axis_name="c", subcore_axis_name="s")

_sc_kernel = functools.partial(
    pl.kernel,
    mesh=_mesh,
    out_type=jax.ShapeDtypeStruct((N, M), jnp.float32),
    scratch_types=[
        pltpu.VMEM((B_PER_W,), jnp.int32),
        pltpu.VMEM((M,), jnp.float32),
        pltpu.VMEM((M,), jnp.float32),
        pltpu.VMEM((LANES,), jnp.float32),
        pltpu.VMEM((LANES,), jnp.float32),
        pltpu.VMEM((NBUF * CHUNK, M), jnp.float32),
        pltpu.SemaphoreType.DMA,
        pltpu.SemaphoreType.DMA,
    ],
)(_sc_body)


def kernel(eigvals, eigvecs, node_indices, lengthscale, variance):
    idx = node_indices.astype(jnp.int32)
    return _sc_kernel(eigvals, eigvecs, idx, lengthscale, variance)
